# baseline (speedup 1.0000x reference)
import functools
import math

import jax
import jax.numpy as jnp
from jax import lax
from jax.experimental import pallas as pl
from jax.experimental.pallas import tpu as pltpu
from jax.experimental.pallas import tpu_sc as plsc

VOCAB = 100000
EMB = 128
SCALE = math.sqrt(EMB)

NC = 2
NS = 16
NW = NC * NS

G = 128
CH = 128
NBUF = 5


def _scale_body(t_ref, o_ref):
    o_ref[...] = t_ref[...] * SCALE


def _scale_table(table):
    rows = table.shape[0]
    blk = 10000
    return pl.pallas_call(
        _scale_body,
        grid=(rows // blk,),
        in_specs=[pl.BlockSpec((blk, EMB), lambda i: (i, 0))],
        out_specs=pl.BlockSpec((blk, EMB), lambda i: (i, 0)),
        out_shape=jax.ShapeDtypeStruct((rows, EMB), jnp.float32),
    )(table)


def _make_sc_gather(n_flat, offset, n_rows):
    b_per_w = n_rows // NW
    ng = b_per_w // G
    mesh = plsc.VectorSubcoreMesh(core_axis_name="c", subcore_axis_name="s")

    gpc = CH // G
    nch = b_per_w // CH

    @functools.partial(
        pl.kernel,
        mesh=mesh,
        out_type=jax.ShapeDtypeStruct((n_flat, EMB), jnp.float32),
        scratch_types=[
            pltpu.VMEM((ng, G), jnp.int32),
            pltpu.VMEM((NBUF, CH, EMB), jnp.float32),
            pltpu.SemaphoreType.DMA,
            pltpu.SemaphoreType.DMA,
        ],
    )
    def sc_gather(table_hbm, idx_hbm, out_hbm, idx_v, buf_v, gsem, osem):
        wid = lax.axis_index("s") * NC + lax.axis_index("c")
        base = offset + wid * b_per_w
        pltpu.sync_copy(idx_hbm.at[wid], idx_v)

        def fire(ch, b):
            for g in range(gpc):
                pltpu.async_copy(
                    table_hbm.at[idx_v.at[ch * gpc + g]],
                    buf_v.at[b, pl.ds(g * G, G)], gsem)

        for b in range(NBUF):
            fire(b, b)

        def step(cc, _):
            c0 = cc * NBUF
            for b in range(NBUF):
                ch = c0 + b
                for g in range(gpc):
                    pltpu.make_async_copy(
                        table_hbm.at[idx_v.at[0]],
                        buf_v.at[b, pl.ds(g * G, G)], gsem).wait()
                pltpu.async_copy(
                    buf_v.at[b], out_hbm.at[pl.ds(base + ch * CH, CH)], osem)
                nch_b = ch + NBUF

                @pl.when(nch_b < nch)
                def _():
                    pltpu.make_async_copy(
                        buf_v.at[b], out_hbm.at[pl.ds(base, CH)], osem).wait()
                    fire(nch_b, b)
            return ()

        lax.fori_loop(0, nch // NBUF, step, (), unroll=False)

        for b in range(NBUF):
            pltpu.make_async_copy(
                buf_v.at[b], out_hbm.at[pl.ds(base, CH)], osem).wait()

    return sc_gather


TC_N = 102400
TC_BLK = 2048


def _tc_gather_body(idx_ref, table_ref, o_ref):
    def row(i, _):
        o_ref[i, :] = table_ref[idx_ref[i], :]
        return ()

    lax.fori_loop(0, TC_BLK, row, (), unroll=8)


def _tc_gather(scaled, idx_tc):
    return pl.pallas_call(
        _tc_gather_body,
        grid=(TC_N // TC_BLK,),
        in_specs=[
            pl.BlockSpec((TC_BLK,), lambda i: (i,),
                         memory_space=pltpu.SMEM),
            pl.BlockSpec(memory_space=pltpu.VMEM),
        ],
        out_specs=pl.BlockSpec((TC_BLK, EMB), lambda i: (i, 0)),
        out_shape=jax.ShapeDtypeStruct((TC_N, EMB), jnp.float32),
    )(idx_tc, scaled)


def kernel(tokens, table):
    b, l = tokens.shape
    n_flat = b * l
    sc_n = n_flat - TC_N
    b_per_w = sc_n // NW
    ng = b_per_w // G
    scaled = _scale_table(table)
    flat = tokens.reshape(-1).astype(jnp.int32)
    out_tc = _tc_gather(scaled, flat[:TC_N])
    idx_sc = flat[TC_N:].reshape(NW, ng, G)
    out_sc = _make_sc_gather(n_flat, TC_N, sc_n)(scaled, idx_sc)
    out = lax.dynamic_update_slice(out_sc, out_tc, (0, 0))
    return out.reshape(b, l, EMB)


# speedup vs baseline: 1.1796x; 1.1796x over previous
import functools
import math

import jax
import jax.numpy as jnp
from jax import lax
from jax.experimental import pallas as pl
from jax.experimental.pallas import tpu as pltpu
from jax.experimental.pallas import tpu_sc as plsc

VOCAB = 100000
EMB = 128
SCALE = math.sqrt(EMB)

NC = 2
NS = 16
NW = NC * NS

G = 128
NBUF = 5
LANES = 16


def _make_sc_gather(n_flat):
    b_per_w = n_flat // NW
    ng = b_per_w // G
    mesh = plsc.VectorSubcoreMesh(core_axis_name="c", subcore_axis_name="s")

    @functools.partial(
        pl.kernel,
        mesh=mesh,
        out_type=jax.ShapeDtypeStruct((n_flat, EMB), jnp.float32),
        scratch_types=[
            pltpu.VMEM((ng, G), jnp.int32),
            pltpu.VMEM((NBUF, G, EMB), jnp.float32),
            pltpu.SemaphoreType.DMA,
            pltpu.SemaphoreType.DMA,
        ],
    )
    def sc_gather(table_hbm, idx_hbm, out_hbm, idx_v, buf_v, gsem, osem):
        wid = lax.axis_index("s") * NC + lax.axis_index("c")
        base = wid * b_per_w
        pltpu.sync_copy(idx_hbm.at[wid], idx_v)

        for b in range(NBUF):
            pltpu.async_copy(table_hbm.at[idx_v.at[b]], buf_v.at[b], gsem)

        def scale_chunk(b):
            def row(r, _):
                for c in range(EMB // LANES):
                    sl = pl.ds(c * LANES, LANES)
                    buf_v[b, r, sl] = buf_v[b, r, sl] * SCALE
                return ()

            lax.fori_loop(0, G, row, (), unroll=2)

        def step(jj, _):
            j0 = jj * NBUF
            for b in range(NBUF):
                j = j0 + b
                pltpu.make_async_copy(
                    table_hbm.at[idx_v.at[0]], buf_v.at[b], gsem).wait()
                scale_chunk(b)
                pltpu.async_copy(
                    buf_v.at[b], out_hbm.at[pl.ds(base + j * G, G)], osem)
                nj = j + NBUF

                @pl.when(nj < ng)
                def _():
                    pltpu.make_async_copy(
                        buf_v.at[b], out_hbm.at[pl.ds(base, G)], osem).wait()
                    pltpu.async_copy(
                        table_hbm.at[idx_v.at[nj]], buf_v.at[b], gsem)
            return ()

        lax.fori_loop(0, ng // NBUF, step, (), unroll=False)

        for b in range(NBUF):
            pltpu.make_async_copy(
                buf_v.at[b], out_hbm.at[pl.ds(base, G)], osem).wait()

    return sc_gather


def kernel(tokens, table):
    b, l = tokens.shape
    n_flat = b * l
    b_per_w = n_flat // NW
    ng = b_per_w // G
    idx = tokens.reshape(NW, ng, G).astype(jnp.int32)
    out = _make_sc_gather(n_flat)(table, idx)
    return out.reshape(b, l, EMB)
